# baseline (speedup 1.0000x reference)
import functools

import jax
import jax.numpy as jnp
import numpy as np
from jax import lax
from jax.experimental import pallas as pl
from jax.experimental.pallas import tpu as pltpu
from jax.experimental.pallas import tpu_sc as plsc

N_PREFIX_HASH_BINS = 100000
MAX_SEQ_LEN = 200
PRIME = (1 << 31) - 1
BINS1 = N_PREFIX_HASH_BINS - 1

_rng = np.random.RandomState(42)
_A = _rng.randint(1, PRIME, size=(MAX_SEQ_LEN,)).astype(np.int64)
_B = int(_rng.randint(0, PRIME))

_PAD_LEN = 208
_A_PAD = np.zeros((_PAD_LEN,), np.int64)
_A_PAD[:MAX_SEQ_LEN] = _A
_A_LO = (_A_PAD & 0xFFFF).astype(np.int32)
_A_HI = (_A_PAD >> 16).astype(np.int32)

_NC, _NS = 2, 16
_NW = _NC * _NS
_NCHUNK = _PAD_LEN // 16


def _make_sc_kernel(batch, seqlen):
    rows_per = batch // _NW
    blk = rows_per * _PAD_LEN
    mesh = plsc.VectorSubcoreMesh(core_axis_name="c", subcore_axis_name="s")

    @functools.partial(
        pl.kernel,
        out_type=jax.ShapeDtypeStruct((batch, 2 * seqlen), jnp.int32),
        mesh=mesh,
        compiler_params=pltpu.CompilerParams(needs_layout_passes=False),
        scratch_types=[
            pltpu.VMEM((blk,), jnp.int32),
            pltpu.VMEM((rows_per, 2 * seqlen), jnp.int32),
            pltpu.VMEM((_PAD_LEN,), jnp.int32),
            pltpu.VMEM((_PAD_LEN,), jnp.int32),
        ],
    )
    def body(seq_hbm, alo_hbm, ahi_hbm, out_hbm, seq_v, out_v, alo_v, ahi_v):
        _U16 = jnp.uint32(0xFFFF)
        _U15 = jnp.uint32(0x7FFF)
        _UP = jnp.uint32(PRIME)
        _INV_BINS1 = jnp.float32(1.0 / BINS1)
        _IBINS1 = jnp.int32(BINS1)
        wid = lax.axis_index("s") * _NC + lax.axis_index("c")
        pltpu.sync_copy(seq_hbm.at[pl.ds(wid * blk, blk)], seq_v)
        pltpu.sync_copy(alo_hbm, alo_v)
        pltpu.sync_copy(ahi_hbm, ahi_v)
        pos0 = lax.iota(jnp.int32, 16)
        zeros16 = pos0 * 0
        cols_c = [jnp.minimum((pos0 + 16 * j) * 2, jnp.int32(2 * seqlen - 2))
                  for j in range(_NCHUNK)]

        _UB0 = jnp.uint32(_B & 0xFFFF)
        _UB1 = jnp.uint32(_B >> 16)

        @plsc.parallel_loop(jnp.int32(0), jnp.int32(rows_per),
                            step=jnp.int32(1), unroll=2)
        def row_body(r):
            base = r * _PAD_LEN
            rfull = jnp.full((16,), r, jnp.int32)
            n = zeros16
            c02 = _UB0
            c1 = _UB1
            for j in range(_NCHUNK):
                a0 = plsc.bitcast(alo_v[pl.ds(16 * j, 16)], jnp.uint32)
                a1 = plsc.bitcast(ahi_v[pl.ds(16 * j, 16)], jnp.uint32)
                msk = None
                lanes_ok = None
                if 16 * (j + 1) > seqlen:
                    lanes_ok = pos0 < jnp.int32(seqlen - 16 * j)
                    msk = lanes_ok
                x_i = seq_v[pl.ds(base + 16 * j, 16)]
                x = plsc.bitcast(x_i, jnp.uint32)
                x0 = x & _U16
                x1 = x >> jnp.uint32(16)
                m00 = a0 * x0
                m10 = a1 * x0
                m01 = a0 * x1
                m11 = a1 * x1
                e02 = (m00 & _U16) + ((m10 >> jnp.uint32(16)) + m11) * jnp.uint32(2)
                e1 = (m00 >> jnp.uint32(16)) + (m10 & _U16) + m01
                l02 = plsc.cumsum(e02) + c02
                l1 = plsc.cumsum(e1) + c1
                c02 = c02 + jnp.sum(e02, dtype=jnp.uint32)
                c1 = c1 + jnp.sum(e1, dtype=jnp.uint32)
                s16v = ((l1 & _U15) << jnp.uint32(16)) + (l1 >> jnp.uint32(15))
                acc = l02 + s16v
                h = (acc & _UP) + (acc >> jnp.uint32(31))
                h = jnp.where(h >= _UP, h - _UP, h)
                hi = plsc.bitcast(h, jnp.int32)
                q = (hi.astype(jnp.float32) * _INV_BINS1).astype(jnp.int32)
                rv = hi - q * _IBINS1
                rv = jnp.where(rv < 0, rv + _IBINS1, rv)
                rv = jnp.where(rv >= _IBINS1, rv - _IBINS1, rv)
                idv = rv + 1
                nzb = x_i != 0
                if lanes_ok is not None:
                    nzb = nzb & lanes_ok
                n = n + plsc.all_reduce_population_count(nzb)
                plsc.store_scatter(out_v, [rfull, cols_c[j]], idv, mask=msk)
                plsc.store_scatter(out_v, [rfull, cols_c[j] + 1], zeros16,
                                   mask=msk)
            last_idx = jnp.clip(n - 1, 0, seqlen - 1)
            last_vec = plsc.load_gather(out_v, [rfull, last_idx * 2])

            def tail_body(k, carry2):
                posk = pos0 + k * 16
                m = (posk >= n) & (posk < jnp.int32(seqlen))
                ck = jnp.minimum(posk * 2, jnp.int32(2 * seqlen - 2))
                plsc.store_scatter(out_v, [rfull, ck], last_vec, mask=m)
                return carry2

            lax.fori_loop(jnp.max(n) // jnp.int32(16), jnp.int32(_NCHUNK),
                          tail_body, jnp.int32(0))
        pltpu.sync_copy(out_v,
                        out_hbm.at[pl.ds(wid * rows_per, rows_per), :])

    return body


def kernel(sequences):
    batch, seqlen = sequences.shape
    x = sequences.astype(jnp.int32)
    xp = jnp.pad(x, ((0, 0), (0, _PAD_LEN - seqlen)))
    nsplit = 4
    part = batch // nsplit
    sc = _make_sc_kernel(part, seqlen)
    alo, ahi = jnp.asarray(_A_LO), jnp.asarray(_A_HI)
    outs = []
    for i in range(nsplit):
        pairs = sc(xp[i * part:(i + 1) * part].reshape(-1), alo, ahi)
        outs.append(jax.lax.bitcast_convert_type(
            pairs.reshape(part, seqlen, 2), jnp.int64))
    return jnp.concatenate(outs, axis=0)


# speedup vs baseline: 1.2382x; 1.2382x over previous
import functools

import jax
import jax.numpy as jnp
import numpy as np
from jax import lax
from jax.experimental import pallas as pl
from jax.experimental.pallas import tpu as pltpu
from jax.experimental.pallas import tpu_sc as plsc

N_PREFIX_HASH_BINS = 100000
MAX_SEQ_LEN = 200
PRIME = (1 << 31) - 1
BINS1 = N_PREFIX_HASH_BINS - 1

_rng = np.random.RandomState(42)
_A = _rng.randint(1, PRIME, size=(MAX_SEQ_LEN,)).astype(np.int64)
_B = int(_rng.randint(0, PRIME))

_PAD_LEN = 208
_A_PAD = np.zeros((_PAD_LEN,), np.int64)
_A_PAD[:MAX_SEQ_LEN] = _A
_A_LO = (_A_PAD & 0xFFFF).astype(np.int32)
_A_HI = (_A_PAD >> 16).astype(np.int32)

_NC, _NS = 2, 16
_NW = _NC * _NS
_NCHUNK = _PAD_LEN // 16


def _make_sc_kernel(batch, seqlen):
    rows_per = batch // _NW
    blk = rows_per * _PAD_LEN
    mesh = plsc.VectorSubcoreMesh(core_axis_name="c", subcore_axis_name="s")

    @functools.partial(
        pl.kernel,
        out_type=jax.ShapeDtypeStruct((batch, 2 * seqlen), jnp.int32),
        mesh=mesh,
        compiler_params=pltpu.CompilerParams(needs_layout_passes=False),
        scratch_types=[
            pltpu.VMEM((blk,), jnp.int32),
            pltpu.VMEM((rows_per, 2 * seqlen), jnp.int32),
            pltpu.VMEM((_PAD_LEN,), jnp.int32),
            pltpu.VMEM((_PAD_LEN,), jnp.int32),
        ],
    )
    def body(seq_hbm, alo_hbm, ahi_hbm, out_hbm, seq_v, out_v, alo_v, ahi_v):
        _U16 = jnp.uint32(0xFFFF)
        _U15 = jnp.uint32(0x7FFF)
        _UP = jnp.uint32(PRIME)
        _INV_BINS1 = jnp.float32(1.0 / BINS1)
        _IBINS1 = jnp.int32(BINS1)
        wid = lax.axis_index("s") * _NC + lax.axis_index("c")
        pltpu.sync_copy(seq_hbm.at[pl.ds(wid * blk, blk)], seq_v)
        pltpu.sync_copy(alo_hbm, alo_v)
        pltpu.sync_copy(ahi_hbm, ahi_v)
        pos0 = lax.iota(jnp.int32, 16)
        zeros16 = pos0 * 0
        cols_c = [jnp.minimum((pos0 + 16 * j) * 2, jnp.int32(2 * seqlen - 2))
                  for j in range(_NCHUNK)]

        _UB0 = jnp.uint32(_B & 0xFFFF)
        _UB1 = jnp.uint32(_B >> 16)

        @plsc.parallel_loop(jnp.int32(0), jnp.int32(rows_per),
                            step=jnp.int32(1), unroll=2)
        def row_body(r):
            base = r * _PAD_LEN
            rfull = jnp.full((16,), r, jnp.int32)
            n = zeros16
            c02 = _UB0
            c1 = _UB1
            for j in range(_NCHUNK):
                a0 = plsc.bitcast(alo_v[pl.ds(16 * j, 16)], jnp.uint32)
                a1 = plsc.bitcast(ahi_v[pl.ds(16 * j, 16)], jnp.uint32)
                msk = None
                lanes_ok = None
                if 16 * (j + 1) > seqlen:
                    lanes_ok = pos0 < jnp.int32(seqlen - 16 * j)
                    msk = lanes_ok
                x_i = seq_v[pl.ds(base + 16 * j, 16)]
                x = plsc.bitcast(x_i, jnp.uint32)
                x0 = x & _U16
                x1 = x >> jnp.uint32(16)
                m00 = a0 * x0
                m10 = a1 * x0
                m01 = a0 * x1
                m11 = a1 * x1
                e02 = (m00 & _U16) + ((m10 >> jnp.uint32(16)) + m11) * jnp.uint32(2)
                e1 = (m00 >> jnp.uint32(16)) + (m10 & _U16) + m01
                l02 = plsc.cumsum(e02) + c02
                l1 = plsc.cumsum(e1) + c1
                c02 = c02 + jnp.sum(e02, dtype=jnp.uint32)
                c1 = c1 + jnp.sum(e1, dtype=jnp.uint32)
                s16v = ((l1 & _U15) << jnp.uint32(16)) + (l1 >> jnp.uint32(15))
                acc = l02 + s16v
                h = (acc & _UP) + (acc >> jnp.uint32(31))
                h = jnp.where(h >= _UP, h - _UP, h)
                hi = plsc.bitcast(h, jnp.int32)
                q = (hi.astype(jnp.float32) * _INV_BINS1).astype(jnp.int32)
                rv = hi - q * _IBINS1
                rv = jnp.where(rv < 0, rv + _IBINS1, rv)
                rv = jnp.where(rv >= _IBINS1, rv - _IBINS1, rv)
                idv = rv + 1
                nzb = x_i != 0
                if lanes_ok is not None:
                    nzb = nzb & lanes_ok
                n = n + plsc.all_reduce_population_count(nzb)
                plsc.store_scatter(out_v, [rfull, cols_c[j]], idv, mask=msk)
                plsc.store_scatter(out_v, [rfull, cols_c[j] + 1], zeros16,
                                   mask=msk)
            last_idx = jnp.clip(n - 1, 0, seqlen - 1)
            last_vec = plsc.load_gather(out_v, [rfull, last_idx * 2])

            def tail_body(k, carry2):
                posk = pos0 + k * 16
                m = (posk >= n) & (posk < jnp.int32(seqlen))
                ck = jnp.minimum(posk * 2, jnp.int32(2 * seqlen - 2))
                plsc.store_scatter(out_v, [rfull, ck], last_vec, mask=m)
                return carry2

            lax.fori_loop(jnp.max(n) // jnp.int32(16), jnp.int32(_NCHUNK),
                          tail_body, jnp.int32(0))
        pltpu.sync_copy(out_v,
                        out_hbm.at[pl.ds(wid * rows_per, rows_per), :])

    return body


def kernel(sequences):
    batch, seqlen = sequences.shape
    x = sequences.astype(jnp.int32)
    xp = jnp.pad(x, ((0, 0), (0, _PAD_LEN - seqlen)))
    nsplit = 2
    part = batch // nsplit
    sc = _make_sc_kernel(part, seqlen)
    alo, ahi = jnp.asarray(_A_LO), jnp.asarray(_A_HI)
    outs = []
    for i in range(nsplit):
        pairs = sc(xp[i * part:(i + 1) * part].reshape(-1), alo, ahi)
        outs.append(jax.lax.bitcast_convert_type(
            pairs.reshape(part, seqlen, 2), jnp.int64))
    return jnp.concatenate(outs, axis=0)
